# Initial kernel scaffold; baseline (speedup 1.0000x reference)
import jax
import jax.numpy as jnp
from jax.experimental import pallas as pl


def kernel(h, u, pos_state, pos_action, a2s_edge_index, a2s_dis, s2s_edge_index, s2s_dis, params):
    raise NotImplementedError("write your pallas kernel here")


# speedup vs baseline: 1.2109x; 1.2109x over previous
import functools

import jax
import jax.numpy as jnp
from jax.experimental import pallas as pl

_N = 10000
_E = 320000
_HID = 32
_MLP = 64

_BN = 400
_BE = 3200


def _prep_body(u_ref, h_ref, uw_ref, ub_ref, hw_ref, hb_ref, up_ref, hp_ref):
    up_ref[...] = u_ref[...] @ uw_ref[...] + ub_ref[...]
    hp_ref[...] = h_ref[...] @ hw_ref[...] + hb_ref[...]


def _edge_body(fa_ref, ug_ref, fs_ref, hg_ref,
               wa1, ba1, wa2, ba2, wa3, ba3,
               ws1, bs1, ws2, bs2, ws3, bs3,
               msga_ref, es_ref, ems_ref):
    ta = jnp.tanh(fa_ref[...] @ wa1[...] + ba1[...])
    ta = jnp.tanh(ta @ wa2[...] + ba2[...])
    gate = jax.nn.sigmoid(ta @ wa3[...] + ba3[...])
    msga_ref[...] = gate * ug_ref[...]
    ts = jnp.tanh(fs_ref[...] @ ws1[...] + bs1[...])
    ts = jnp.tanh(ts @ ws2[...] + bs2[...])
    e = jnp.exp(ts @ ws3[...] + bs3[...])
    es_ref[...] = e
    ems_ref[...] = e * hg_ref[...]


def _final_body(h_ref, su_ref, den_ref, sem_ref, ps_ref,
                cw1, cb1, cw2, cb2, cw3, cb3,
                kw1, kb1, kw2, kb2, kw3, kb3, out_ref):
    den = den_ref[...]
    sum_h = jnp.where(den != 0.0, sem_ref[...] / den, 0.0)
    inp = jnp.concatenate(
        [h_ref[...], su_ref[...], sum_h,
         jnp.zeros((_BN, 128 - 3 * _HID), jnp.float32)], axis=1)
    m = jnp.tanh(ps_ref[...] @ cw1[...] + cb1[...])
    m = jnp.tanh(m @ cw2[...] + cb2[...])
    coef = m @ cw3[...] + cb3[...]
    k = jnp.tanh(ps_ref[...] @ kw1[...] + kb1[...])
    k = jnp.tanh(k @ kw2[...] + kb2[...])
    const = k @ kw3[...] + kb3[...]
    prod = coef.reshape(_BN, _HID, 128) * inp[:, None, :]
    out_ref[...] = jnp.sum(prod, axis=-1) + const


def _full(shape):
    return pl.BlockSpec(shape, lambda i: (0, 0))


def kernel(h, u, pos_state, pos_action, a2s_edge_index, a2s_dis,
           s2s_edge_index, s2s_dis, params):
    p = params
    f32 = jnp.float32

    def row(b):
        return b.reshape(1, -1).astype(f32)

    up, hp = pl.pallas_call(
        _prep_body,
        grid=(_N // _BN,),
        in_specs=[
            pl.BlockSpec((_BN, 128), lambda i: (i, 0)),
            pl.BlockSpec((_BN, _HID), lambda i: (i, 0)),
            _full((128, _HID)), _full((1, _HID)),
            _full((_HID, _HID)), _full((1, _HID)),
        ],
        out_specs=[
            pl.BlockSpec((_BN, _HID), lambda i: (i, 0)),
            pl.BlockSpec((_BN, _HID), lambda i: (i, 0)),
        ],
        out_shape=[
            jax.ShapeDtypeStruct((_N, _HID), f32),
            jax.ShapeDtypeStruct((_N, _HID), f32),
        ],
    )(u, h, p["u2h_u_W"], row(p["u2h_u_b"]), p["h2h_h_W"], row(p["h2h_h_b"]))

    src_a, dst_a = a2s_edge_index[0], a2s_edge_index[1]
    src_s, dst_s = s2s_edge_index[0], s2s_edge_index[1]
    zpad = jnp.zeros((_E, 3), f32)
    feat_a = jnp.concatenate(
        [pos_action[src_a], pos_state[dst_a], a2s_dis, zpad], axis=1)
    feat_s = jnp.concatenate(
        [pos_state[src_s], pos_state[dst_s], s2s_dis, zpad], axis=1)
    ug = up[src_a]
    hg = hp[src_s]

    def pad8(w):
        return jnp.pad(w, ((0, 8 - w.shape[0]), (0, 0)))

    wspecs = []
    wvals = []
    for pre, first_pad in (("u2h_dis", True), ("h2h_dis", True)):
        w1 = p[pre + "_W1"]
        wvals += [pad8(w1), row(p[pre + "_b1"]),
                  p[pre + "_W2"], row(p[pre + "_b2"]),
                  p[pre + "_W3"], row(p[pre + "_b3"])]
        wspecs += [_full((8, _MLP)), _full((1, _MLP)),
                   _full((_MLP, _MLP)), _full((1, _MLP)),
                   _full((_MLP, _HID)), _full((1, _HID))]

    msga, es, ems = pl.pallas_call(
        _edge_body,
        grid=(_E // _BE,),
        in_specs=[
            pl.BlockSpec((_BE, 8), lambda i: (i, 0)),
            pl.BlockSpec((_BE, _HID), lambda i: (i, 0)),
            pl.BlockSpec((_BE, 8), lambda i: (i, 0)),
            pl.BlockSpec((_BE, _HID), lambda i: (i, 0)),
        ] + wspecs,
        out_specs=[
            pl.BlockSpec((_BE, _HID), lambda i: (i, 0)),
            pl.BlockSpec((_BE, _HID), lambda i: (i, 0)),
            pl.BlockSpec((_BE, _HID), lambda i: (i, 0)),
        ],
        out_shape=[
            jax.ShapeDtypeStruct((_E, _HID), f32),
            jax.ShapeDtypeStruct((_E, _HID), f32),
            jax.ShapeDtypeStruct((_E, _HID), f32),
        ],
    )(feat_a, ug, feat_s, hg, *wvals)

    sum_u = jax.ops.segment_sum(msga, dst_a, num_segments=_N)
    den = jax.ops.segment_sum(es, dst_s, num_segments=_N)
    sem = jax.ops.segment_sum(ems, dst_s, num_segments=_N)

    cw3 = p["coef_W3"].reshape(_MLP, _HID, 3 * _HID)
    cw3 = jnp.pad(cw3, ((0, 0), (0, 0), (0, 128 - 3 * _HID)))
    cw3 = cw3.reshape(_MLP, _HID * 128)
    cb3 = p["coef_b3"].reshape(_HID, 3 * _HID)
    cb3 = jnp.pad(cb3, ((0, 0), (0, 128 - 3 * _HID))).reshape(1, _HID * 128)

    out = pl.pallas_call(
        _final_body,
        grid=(_N // _BN,),
        in_specs=[
            pl.BlockSpec((_BN, _HID), lambda i: (i, 0)),
            pl.BlockSpec((_BN, _HID), lambda i: (i, 0)),
            pl.BlockSpec((_BN, _HID), lambda i: (i, 0)),
            pl.BlockSpec((_BN, _HID), lambda i: (i, 0)),
            pl.BlockSpec((_BN, 2), lambda i: (i, 0)),
            _full((2, _MLP)), _full((1, _MLP)),
            _full((_MLP, _MLP)), _full((1, _MLP)),
            _full((_MLP, _HID * 128)), _full((1, _HID * 128)),
            _full((2, _MLP)), _full((1, _MLP)),
            _full((_MLP, _MLP)), _full((1, _MLP)),
            _full((_MLP, _HID)), _full((1, _HID)),
        ],
        out_specs=[pl.BlockSpec((_BN, _HID), lambda i: (i, 0))],
        out_shape=[jax.ShapeDtypeStruct((_N, _HID), f32)],
    )(h, sum_u, den, sem, pos_state,
      p["coef_W1"], row(p["coef_b1"]), p["coef_W2"], row(p["coef_b2"]),
      cw3, cb3,
      p["const_W1"], row(p["const_b1"]), p["const_W2"], row(p["const_b2"]),
      p["const_W3"], row(p["const_b3"]))[0]
    return out


# speedup vs baseline: 6.1211x; 5.0549x over previous
import jax
import jax.numpy as jnp
from jax import lax
from jax.experimental import pallas as pl
from jax.experimental.pallas import tpu as pltpu
from jax.experimental.pallas import tpu_sc as plsc

_N = 10000
_E = 320000
_HID = 32
_MLP = 64

_BN = 400
_BE = 2048

_EP = 327680
_EPH = _EP // 2
_NW = 32

_SB = 32
_NSB = 8
_BLK = _SB * _NSB
_NTRIP = _EPH // (_NW * _BLK)
_NP = 10240
_NTROW = 640
_CHUNKS = ((0, 256), (256, 256), (512, 128))

_GSB = 32
_GNSB = 8
_GBLK = _GSB * _GNSB
_GNTRIP = _EPH // (_NW * _GBLK)
_GCHUNKS = ((0, 256), (256, 256), (512, 128))


def _prep_body(u_ref, h_ref, pa_ref, ps_ref,
               uw_ref, ub_ref, hw_ref, hb_ref,
               was_ref, bas_ref, wad_ref, wss_ref, bss_ref, wsd_ref,
               tsa_ref, tda_ref, tss_ref, tds_ref):
    z32 = jnp.zeros((_BN, _HID), jnp.float32)
    z64 = jnp.zeros((_BN, 2 * _HID), jnp.float32)
    up = u_ref[...] @ uw_ref[...] + ub_ref[...]
    hp = h_ref[...] @ hw_ref[...] + hb_ref[...]
    pa = pa_ref[...]
    ps = ps_ref[...]
    tsa_ref[...] = jnp.concatenate(
        [pa @ was_ref[...] + bas_ref[...], up, z32], axis=1)
    tda_ref[...] = jnp.concatenate([ps @ wad_ref[...], z64], axis=1)
    tss_ref[...] = jnp.concatenate(
        [ps @ wss_ref[...] + bss_ref[...], hp, z32], axis=1)
    tds_ref[...] = jnp.concatenate([ps @ wsd_ref[...], z64], axis=1)


def _make_edge_body(base):
    def _edge_body(xsa_ref, xda_ref, disa_ref,
                   xss_ref, xds_ref, diss_ref,
                   wa1z, ws1z, w2bd, b2c, w3bd, b3c,
                   msga_ref, ems_ref):
        _edge_compute(base, xsa_ref, xda_ref, disa_ref,
                      xss_ref, xds_ref, diss_ref,
                      wa1z, ws1z, w2bd, b2c, w3bd, b3c,
                      msga_ref, ems_ref)
    return _edge_body


def _edge_compute(base, xsa_ref, xda_ref, disa_ref,
                  xss_ref, xds_ref, diss_ref,
                  wa1z, ws1z, w2bd, b2c, w3bd, b3c,
                  msga_ref, ems_ref):
    rowid = base + _BE * pl.program_id(0) + lax.broadcasted_iota(
        jnp.int32, (_BE, 1), 0)
    live = jnp.where(rowid < _E, 1.0, 0.0)
    z32 = jnp.zeros((_BE, _HID), jnp.float32)
    xsa = xsa_ref[...]
    xss = xss_ref[...]
    pre_a = xsa[:, :_MLP] + xda_ref[:, :_MLP] + disa_ref[...] * wa1z[...]
    pre_s = xss[:, :_MLP] + xds_ref[:, :_MLP] + diss_ref[...] * ws1z[...]
    t = jnp.tanh(jnp.concatenate([pre_a, pre_s], axis=1))
    t = jnp.tanh(t @ w2bd[...] + b2c[...])
    y = t @ w3bd[...] + b3c[...]
    gate = jax.nn.sigmoid(y[:, :_HID])
    e = live * jnp.exp(y[:, _HID:])
    msga_ref[...] = jnp.concatenate(
        [live * (gate * xsa[:, _MLP:_MLP + _HID]), z32, z32, z32], axis=1)
    ems_ref[...] = jnp.concatenate(
        [z32, e, e * xss[:, _MLP:_MLP + _HID], z32], axis=1)


def _sc_scatter_body(dsta_hbm, msga_hbm, dsts_hbm, msgs_hbm, z_hbm,
                     part_hbm, idx_v, msg_v, acc, sem):
    c = lax.axis_index("c")
    s = lax.axis_index("s")
    g = c * 16 + s
    r0 = s * _NTROW
    pltpu.sync_copy(z_hbm, msg_v)
    for off, ln in _CHUNKS:
        pltpu.sync_copy(msg_v.at[pl.ds(0, ln)], acc.at[pl.ds(r0 + off, ln)])
    plsc.subcore_barrier()

    for dst_hbm, msg_hbm in ((dsta_hbm, msga_hbm), (dsts_hbm, msgs_hbm)):
        def blk(b, carry, dst_hbm=dst_hbm, msg_hbm=msg_hbm):
            t = g + b * _NW
            pltpu.sync_copy(dst_hbm.at[pl.ds(t * _NSB, _NSB)], idx_v)
            pltpu.sync_copy(msg_hbm.at[pl.ds(t * _BLK, _BLK)], msg_v)
            cps = []
            for j in range(_NSB):
                cps.append(pltpu.async_copy(
                    msg_v.at[pl.ds(j * _SB, _SB)],
                    acc.at[idx_v.at[j]], sem, add=True))
            for cp in cps:
                cp.wait()
            return carry

        lax.fori_loop(0, _NTRIP, blk, 0)

    plsc.subcore_barrier()
    for off, ln in _CHUNKS:
        pltpu.sync_copy(acc.at[pl.ds(r0 + off, ln)], msg_v.at[pl.ds(0, ln)])
        pltpu.sync_copy(msg_v.at[pl.ds(0, ln)],
                        part_hbm.at[c, pl.ds(r0 + off, ln)])


def _sc_scatter(dsta2d, msga, dsts2d, msgs):
    f32 = jnp.float32
    z = jnp.zeros((_BLK, 128), f32)
    return pl.kernel(
        _sc_scatter_body,
        mesh=plsc.VectorSubcoreMesh(core_axis_name="c", subcore_axis_name="s"),
        out_type=jax.ShapeDtypeStruct((2, _NP, 128), f32),
        scratch_types=[
            pltpu.VMEM((_NSB, _SB), jnp.int32),
            pltpu.VMEM((_BLK, 128), f32),
            pltpu.VMEM_SHARED((_NP, 128), f32),
            pltpu.SemaphoreType.DMA,
        ],
    )(dsta2d, msga, dsts2d, msgs, z)


def _sc_gather4_body(t1, t2, t3, t4, i1, i2, i3, i4, o1, o2, o3, o4,
                     idx_v, row_v, tab_sp, sem):
    c = lax.axis_index("c")
    s = lax.axis_index("s")
    g = c * 16 + s
    r0 = s * _NTROW
    for tab_hbm, idx_hbm, out_hbm in (
            (t1, i1, o1), (t2, i2, o2), (t3, i3, o3), (t4, i4, o4)):
        for off, ln in _GCHUNKS:
            pltpu.sync_copy(tab_hbm.at[pl.ds(r0 + off, ln)],
                            row_v.at[pl.ds(0, ln)])
            pltpu.sync_copy(row_v.at[pl.ds(0, ln)],
                            tab_sp.at[pl.ds(r0 + off, ln)])
        plsc.subcore_barrier()

        def blk(b, carry, idx_hbm=idx_hbm, out_hbm=out_hbm):
            t = g + b * _NW
            pltpu.sync_copy(idx_hbm.at[pl.ds(t * _GNSB, _GNSB)], idx_v)
            cps = []
            for j in range(_GNSB):
                cps.append(pltpu.async_copy(
                    tab_sp.at[idx_v.at[j]],
                    row_v.at[pl.ds(j * _GSB, _GSB)], sem))
            for cp in cps:
                cp.wait()
            pltpu.sync_copy(row_v, out_hbm.at[pl.ds(t * _GBLK, _GBLK)])
            return carry

        lax.fori_loop(0, _GNTRIP, blk, 0)
        plsc.subcore_barrier()


def _sc_gather4(tabs, idxs):
    f32 = jnp.float32
    return pl.kernel(
        _sc_gather4_body,
        mesh=plsc.VectorSubcoreMesh(core_axis_name="c", subcore_axis_name="s"),
        out_type=[jax.ShapeDtypeStruct((_EPH, 128), f32)] * 4,
        scratch_types=[
            pltpu.VMEM((_GNSB, _GSB), jnp.int32),
            pltpu.VMEM((_GBLK, 128), f32),
            pltpu.VMEM_SHARED((_NP, 128), f32),
            pltpu.SemaphoreType.DMA,
        ],
    )(*tabs, *idxs)


def _final_body(h_ref, part_ref, part2_ref, ps_ref,
                cw1, cb1, cw2, cb2, cw3, cb3, rsel, fsel,
                kw1, kb1, kw2, kb2, kw3, kb3, out_ref):
    pp = (part_ref[0] + part_ref[1]) + (part2_ref[0] + part2_ref[1])
    su = pp[:, :_HID]
    den = pp[:, _HID:2 * _HID]
    sum_h = jnp.where(den != 0.0, pp[:, 2 * _HID:3 * _HID] / den, 0.0)
    inp = jnp.concatenate(
        [h_ref[...], su, sum_h,
         jnp.zeros((_BN, 128 - 3 * _HID), jnp.float32)], axis=1)
    m = jnp.tanh(ps_ref[...] @ cw1[...] + cb1[...])
    m = jnp.tanh(m @ cw2[...] + cb2[...])
    k = jnp.tanh(ps_ref[...] @ kw1[...] + kb1[...])
    k = jnp.tanh(k @ kw2[...] + kb2[...])
    const = k @ kw3[...] + kb3[...]
    yy = inp @ cw3[...]
    yb = inp @ cb3[...]
    p2 = yy * (m @ rsel[...])
    q = jnp.sum(p2.reshape(_BN, 16, 128), axis=1)
    out_ref[...] = q @ fsel[...] + yb + const


def _full(shape):
    return pl.BlockSpec(shape, lambda i: (0, 0))


def kernel(h, u, pos_state, pos_action, a2s_edge_index, a2s_dis,
           s2s_edge_index, s2s_dis, params):
    p = params
    f32 = jnp.float32

    def row(b):
        return b.reshape(1, -1).astype(f32)

    wa1 = p["u2h_dis_W1"]
    ws1 = p["h2h_dis_W1"]
    tsa, tda, tss, tds = pl.pallas_call(
        _prep_body,
        grid=(_N // _BN,),
        in_specs=[
            pl.BlockSpec((_BN, 128), lambda i: (i, 0)),
            pl.BlockSpec((_BN, _HID), lambda i: (i, 0)),
            pl.BlockSpec((_BN, 2), lambda i: (i, 0)),
            pl.BlockSpec((_BN, 2), lambda i: (i, 0)),
            _full((128, _HID)), _full((1, _HID)),
            _full((_HID, _HID)), _full((1, _HID)),
            _full((2, _MLP)), _full((1, _MLP)), _full((2, _MLP)),
            _full((2, _MLP)), _full((1, _MLP)), _full((2, _MLP)),
        ],
        out_specs=[pl.BlockSpec((_BN, 128), lambda i: (i, 0))] * 4,
        out_shape=[jax.ShapeDtypeStruct((_N, 128), f32)] * 4,
    )(u, h, pos_action, pos_state,
      p["u2h_u_W"], row(p["u2h_u_b"]), p["h2h_h_W"], row(p["h2h_h_b"]),
      wa1[0:2], row(p["u2h_dis_b1"]), wa1[2:4],
      ws1[0:2], row(p["h2h_dis_b1"]), ws1[2:4])

    pad_e = (0, _EP - _E)
    src_a = jnp.pad(a2s_edge_index[0], pad_e)
    dst_a = jnp.pad(a2s_edge_index[1], pad_e)
    src_s = jnp.pad(s2s_edge_index[0], pad_e)
    dst_s = jnp.pad(s2s_edge_index[1], pad_e)
    disa = jnp.pad(a2s_dis, (pad_e, (0, 0)))
    diss = jnp.pad(s2s_dis, (pad_e, (0, 0)))
    tabs = [jnp.pad(t, ((0, _NP - _N), (0, 0))) for t in (tsa, tda, tss, tds)]

    w2bd = jnp.zeros((2 * _MLP, 2 * _MLP), f32)
    w2bd = w2bd.at[:_MLP, :_MLP].set(p["u2h_dis_W2"])
    w2bd = w2bd.at[_MLP:, _MLP:].set(p["h2h_dis_W2"])
    b2c = jnp.concatenate([p["u2h_dis_b2"], p["h2h_dis_b2"]]).reshape(1, -1)
    w3bd = jnp.zeros((2 * _MLP, 2 * _HID), f32)
    w3bd = w3bd.at[:_MLP, :_HID].set(p["u2h_dis_W3"])
    w3bd = w3bd.at[_MLP:, _HID:].set(p["h2h_dis_W3"])
    b3c = jnp.concatenate([p["u2h_dis_b3"], p["h2h_dis_b3"]]).reshape(1, -1)
    wvals = [p["u2h_dis_W1"][4:5], p["h2h_dis_W1"][4:5],
             w2bd, b2c, w3bd, b3c]
    wspecs = [_full((1, _MLP)), _full((1, _MLP)),
              _full((2 * _MLP, 2 * _MLP)), _full((1, 2 * _MLP)),
              _full((2 * _MLP, 2 * _HID)), _full((1, 2 * _HID))]

    edge_spec = [
        pl.BlockSpec((_BE, 128), lambda i: (i, 0)),
        pl.BlockSpec((_BE, 128), lambda i: (i, 0)),
        pl.BlockSpec((_BE, 1), lambda i: (i, 0)),
    ]

    parts = []
    for h0 in (0, _EPH):
        sl = slice(h0, h0 + _EPH)
        idxs = [a[sl].reshape(_EPH // _GSB, _GSB)
                for a in (src_a, dst_a, src_s, dst_s)]
        xsa, xda, xss, xds = _sc_gather4(tabs, idxs)
        msga, ems = pl.pallas_call(
            _make_edge_body(h0),
            grid=(_EPH // _BE,),
            in_specs=edge_spec + edge_spec + wspecs,
            out_specs=[
                pl.BlockSpec((_BE, 128), lambda i: (i, 0)),
                pl.BlockSpec((_BE, 128), lambda i: (i, 0)),
            ],
            out_shape=[
                jax.ShapeDtypeStruct((_EPH, 128), f32),
                jax.ShapeDtypeStruct((_EPH, 128), f32),
            ],
        )(xsa, xda, disa[sl], xss, xds, diss[sl], *wvals)
        parts.append(_sc_scatter(
            dst_a[sl].reshape(_EPH // _SB, _SB), msga,
            dst_s[sl].reshape(_EPH // _SB, _SB), ems))
    part1, part2 = parts

    cw3 = p["coef_W3"].reshape(_MLP, _HID, 3 * _HID)
    cw3 = jnp.pad(cw3, ((0, 0), (0, 0), (0, 128 - 3 * _HID)))
    cw3 = cw3.transpose(2, 0, 1).reshape(128, _MLP * _HID)
    cb3 = p["coef_b3"].reshape(_HID, 3 * _HID)
    cb3 = jnp.pad(cb3, ((0, 0), (0, 128 - 3 * _HID))).T
    sel = jnp.arange(_MLP * _HID)
    rsel = jnp.zeros((_MLP, _MLP * _HID), f32).at[sel // _HID, sel].set(1.0)
    fsel = jnp.zeros((128, _HID), f32).at[
        jnp.arange(128), jnp.arange(128) % _HID].set(1.0)

    out = pl.pallas_call(
        _final_body,
        grid=(_N // _BN,),
        in_specs=[
            pl.BlockSpec((_BN, _HID), lambda i: (i, 0)),
            pl.BlockSpec((2, _BN, 128), lambda i: (0, i, 0)),
            pl.BlockSpec((2, _BN, 128), lambda i: (0, i, 0)),
            pl.BlockSpec((_BN, 2), lambda i: (i, 0)),
            _full((2, _MLP)), _full((1, _MLP)),
            _full((_MLP, _MLP)), _full((1, _MLP)),
            _full((128, _MLP * _HID)), _full((128, _HID)),
            _full((_MLP, _MLP * _HID)), _full((128, _HID)),
            _full((2, _MLP)), _full((1, _MLP)),
            _full((_MLP, _MLP)), _full((1, _MLP)),
            _full((_MLP, _HID)), _full((1, _HID)),
        ],
        out_specs=[pl.BlockSpec((_BN, _HID), lambda i: (i, 0))],
        out_shape=[jax.ShapeDtypeStruct((_N, _HID), f32)],
    )(h, part1, part2, pos_state,
      p["coef_W1"], row(p["coef_b1"]), p["coef_W2"], row(p["coef_b2"]),
      cw3, cb3, rsel, fsel,
      p["const_W1"], row(p["const_b1"]), p["const_W2"], row(p["const_b2"]),
      p["const_W3"], row(p["const_b3"]))[0]
    return out


# speedup vs baseline: 6.2138x; 1.0151x over previous
import jax
import jax.numpy as jnp
from jax import lax
from jax.experimental import pallas as pl
from jax.experimental.pallas import tpu as pltpu
from jax.experimental.pallas import tpu_sc as plsc

_N = 10000
_E = 320000
_HID = 32
_MLP = 64

_BN = 400
_BE = 2048

_EP = 327680
_EPH = _EP // 2
_NW = 32

_SB = 32
_NSB = 8
_BLK = _SB * _NSB
_NTRIP = _EPH // (_NW * _BLK)
_NP = 10240
_NTROW = 640
_CHUNKS = ((0, 256), (256, 256), (512, 128))

_GSB = 32
_GNSB = 8
_GBLK = _GSB * _GNSB
_GNTRIP = _EPH // (_NW * _GBLK)
_GCHUNKS = ((0, 256), (256, 256), (512, 128))


def _prep_body(u_ref, h_ref, pa_ref, ps_ref,
               uw_ref, ub_ref, hw_ref, hb_ref,
               was_ref, bas_ref, wad_ref, wss_ref, bss_ref, wsd_ref,
               tsa_ref, tda_ref, tss_ref, tds_ref):
    z32 = jnp.zeros((_BN, _HID), jnp.float32)
    z64 = jnp.zeros((_BN, 2 * _HID), jnp.float32)
    up = u_ref[...] @ uw_ref[...] + ub_ref[...]
    hp = h_ref[...] @ hw_ref[...] + hb_ref[...]
    pa = pa_ref[...]
    ps = ps_ref[...]
    tsa_ref[...] = jnp.concatenate(
        [pa @ was_ref[...] + bas_ref[...], up, z32], axis=1)
    tda_ref[...] = jnp.concatenate([ps @ wad_ref[...], z64], axis=1)
    tss_ref[...] = jnp.concatenate(
        [ps @ wss_ref[...] + bss_ref[...], hp, z32], axis=1)
    tds_ref[...] = jnp.concatenate([ps @ wsd_ref[...], z64], axis=1)


def _make_edge_body(base):
    def _edge_body(xsa_ref, xda_ref, disa_ref,
                   xss_ref, xds_ref, diss_ref,
                   wa1z, ws1z, w2bd, b2c, w3bd, b3c,
                   msga_ref, ems_ref):
        _edge_compute(base, xsa_ref, xda_ref, disa_ref,
                      xss_ref, xds_ref, diss_ref,
                      wa1z, ws1z, w2bd, b2c, w3bd, b3c,
                      msga_ref, ems_ref)
    return _edge_body


def _edge_compute(base, xsa_ref, xda_ref, disa_ref,
                  xss_ref, xds_ref, diss_ref,
                  wa1z, ws1z, w2bd, b2c, w3bd, b3c,
                  msga_ref, ems_ref):
    rowid = base + _BE * pl.program_id(0) + lax.broadcasted_iota(
        jnp.int32, (_BE, 1), 0)
    live = jnp.where(rowid < _E, 1.0, 0.0)
    z32 = jnp.zeros((_BE, _HID), jnp.float32)
    xsa = xsa_ref[...]
    xss = xss_ref[...]
    pre_a = xsa[:, :_MLP] + xda_ref[:, :_MLP] + disa_ref[...] * wa1z[...]
    pre_s = xss[:, :_MLP] + xds_ref[:, :_MLP] + diss_ref[...] * ws1z[...]
    t = jnp.tanh(jnp.concatenate([pre_a, pre_s], axis=1))
    t = jnp.tanh(t @ w2bd[...] + b2c[...])
    y = t @ w3bd[...] + b3c[...]
    gate = jax.nn.sigmoid(y[:, :_HID])
    e = live * jnp.exp(y[:, _HID:])
    msga_ref[...] = jnp.concatenate(
        [live * (gate * xsa[:, _MLP:_MLP + _HID]), z32, z32, z32], axis=1)
    ems_ref[...] = jnp.concatenate(
        [z32, e, e * xss[:, _MLP:_MLP + _HID], z32], axis=1)


def _sc_scatter_body(dsta_hbm, msga_hbm, dsts_hbm, msgs_hbm, z_hbm,
                     part_hbm, idx_v, msg_v, acc, sem):
    c = lax.axis_index("c")
    s = lax.axis_index("s")
    g = c * 16 + s
    r0 = s * _NTROW
    pltpu.sync_copy(z_hbm, msg_v)
    for off, ln in _CHUNKS:
        pltpu.sync_copy(msg_v.at[pl.ds(0, ln)], acc.at[pl.ds(r0 + off, ln)])
    plsc.subcore_barrier()

    for dst_hbm, msg_hbm in ((dsta_hbm, msga_hbm), (dsts_hbm, msgs_hbm)):
        def blk(b, carry, dst_hbm=dst_hbm, msg_hbm=msg_hbm):
            t = g + b * _NW
            c1 = pltpu.async_copy(dst_hbm.at[pl.ds(t * _NSB, _NSB)],
                                  idx_v, sem)
            c2 = pltpu.async_copy(msg_hbm.at[pl.ds(t * _BLK, _BLK)],
                                  msg_v, sem)
            c1.wait()
            c2.wait()
            cps = []
            for j in range(_NSB):
                cps.append(pltpu.async_copy(
                    msg_v.at[pl.ds(j * _SB, _SB)],
                    acc.at[idx_v.at[j]], sem, add=True))
            for cp in cps:
                cp.wait()
            return carry

        lax.fori_loop(0, _NTRIP, blk, 0)

    plsc.subcore_barrier()
    for off, ln in _CHUNKS:
        pltpu.sync_copy(acc.at[pl.ds(r0 + off, ln)], msg_v.at[pl.ds(0, ln)])
        pltpu.sync_copy(msg_v.at[pl.ds(0, ln)],
                        part_hbm.at[c, pl.ds(r0 + off, ln)])


def _sc_scatter(dsta2d, msga, dsts2d, msgs):
    f32 = jnp.float32
    z = jnp.zeros((_BLK, 128), f32)
    return pl.kernel(
        _sc_scatter_body,
        mesh=plsc.VectorSubcoreMesh(core_axis_name="c", subcore_axis_name="s"),
        out_type=jax.ShapeDtypeStruct((2, _NP, 128), f32),
        scratch_types=[
            pltpu.VMEM((_NSB, _SB), jnp.int32),
            pltpu.VMEM((_BLK, 128), f32),
            pltpu.VMEM_SHARED((_NP, 128), f32),
            pltpu.SemaphoreType.DMA,
        ],
    )(dsta2d, msga, dsts2d, msgs, z)


def _sc_gather4_body(t1, t2, t3, t4, i1, i2, i3, i4, o1, o2, o3, o4,
                     idx_v, row_v, tab_sp, sem, osem):
    c = lax.axis_index("c")
    s = lax.axis_index("s")
    g = c * 16 + s
    r0 = s * _NTROW
    for tab_hbm, idx_hbm, out_hbm in (
            (t1, i1, o1), (t2, i2, o2), (t3, i3, o3), (t4, i4, o4)):
        for off, ln in _GCHUNKS:
            pltpu.sync_copy(tab_hbm.at[pl.ds(r0 + off, ln)],
                            row_v.at[pl.ds(0, ln)])
            pltpu.sync_copy(row_v.at[pl.ds(0, ln)],
                            tab_sp.at[pl.ds(r0 + off, ln)])
        plsc.subcore_barrier()

        def blk(b, carry, idx_hbm=idx_hbm, out_hbm=out_hbm):
            t = g + b * _NW
            pltpu.sync_copy(idx_hbm.at[pl.ds(t * _GNSB, _GNSB)], idx_v)

            @pl.when(b > 0)
            def _drain():
                pltpu.make_async_copy(
                    row_v, out_hbm.at[pl.ds(g * _GBLK, _GBLK)], osem).wait()

            cps = []
            for j in range(_GNSB):
                cps.append(pltpu.async_copy(
                    tab_sp.at[idx_v.at[j]],
                    row_v.at[pl.ds(j * _GSB, _GSB)], sem))
            for cp in cps:
                cp.wait()
            pltpu.async_copy(row_v, out_hbm.at[pl.ds(t * _GBLK, _GBLK)], osem)
            return carry

        lax.fori_loop(0, _GNTRIP, blk, 0)
        pltpu.make_async_copy(
            row_v, out_hbm.at[pl.ds(g * _GBLK, _GBLK)], osem).wait()
        plsc.subcore_barrier()


def _sc_gather4(tabs, idxs):
    f32 = jnp.float32
    return pl.kernel(
        _sc_gather4_body,
        mesh=plsc.VectorSubcoreMesh(core_axis_name="c", subcore_axis_name="s"),
        out_type=[jax.ShapeDtypeStruct((_EPH, 128), f32)] * 4,
        scratch_types=[
            pltpu.VMEM((_GNSB, _GSB), jnp.int32),
            pltpu.VMEM((_GBLK, 128), f32),
            pltpu.VMEM_SHARED((_NP, 128), f32),
            pltpu.SemaphoreType.DMA,
            pltpu.SemaphoreType.DMA,
        ],
    )(*tabs, *idxs)


def _final_body(h_ref, part_ref, part2_ref, ps_ref,
                cw1, cb1, cw2, cb2, cw3, cb3, rsel, fsel,
                kw1, kb1, kw2, kb2, kw3, kb3, out_ref):
    pp = (part_ref[0] + part_ref[1]) + (part2_ref[0] + part2_ref[1])
    su = pp[:, :_HID]
    den = pp[:, _HID:2 * _HID]
    sum_h = jnp.where(den != 0.0, pp[:, 2 * _HID:3 * _HID] / den, 0.0)
    inp = jnp.concatenate(
        [h_ref[...], su, sum_h,
         jnp.zeros((_BN, 128 - 3 * _HID), jnp.float32)], axis=1)
    m = jnp.tanh(ps_ref[...] @ cw1[...] + cb1[...])
    m = jnp.tanh(m @ cw2[...] + cb2[...])
    k = jnp.tanh(ps_ref[...] @ kw1[...] + kb1[...])
    k = jnp.tanh(k @ kw2[...] + kb2[...])
    const = k @ kw3[...] + kb3[...]
    yy = inp @ cw3[...]
    yb = inp @ cb3[...]
    p2 = yy * (m @ rsel[...])
    q = jnp.sum(p2.reshape(_BN, 16, 128), axis=1)
    out_ref[...] = q @ fsel[...] + yb + const


def _full(shape):
    return pl.BlockSpec(shape, lambda i: (0, 0))


def kernel(h, u, pos_state, pos_action, a2s_edge_index, a2s_dis,
           s2s_edge_index, s2s_dis, params):
    p = params
    f32 = jnp.float32

    def row(b):
        return b.reshape(1, -1).astype(f32)

    wa1 = p["u2h_dis_W1"]
    ws1 = p["h2h_dis_W1"]
    tsa, tda, tss, tds = pl.pallas_call(
        _prep_body,
        grid=(_N // _BN,),
        in_specs=[
            pl.BlockSpec((_BN, 128), lambda i: (i, 0)),
            pl.BlockSpec((_BN, _HID), lambda i: (i, 0)),
            pl.BlockSpec((_BN, 2), lambda i: (i, 0)),
            pl.BlockSpec((_BN, 2), lambda i: (i, 0)),
            _full((128, _HID)), _full((1, _HID)),
            _full((_HID, _HID)), _full((1, _HID)),
            _full((2, _MLP)), _full((1, _MLP)), _full((2, _MLP)),
            _full((2, _MLP)), _full((1, _MLP)), _full((2, _MLP)),
        ],
        out_specs=[pl.BlockSpec((_BN, 128), lambda i: (i, 0))] * 4,
        out_shape=[jax.ShapeDtypeStruct((_N, 128), f32)] * 4,
    )(u, h, pos_action, pos_state,
      p["u2h_u_W"], row(p["u2h_u_b"]), p["h2h_h_W"], row(p["h2h_h_b"]),
      wa1[0:2], row(p["u2h_dis_b1"]), wa1[2:4],
      ws1[0:2], row(p["h2h_dis_b1"]), ws1[2:4])

    pad_e = (0, _EP - _E)
    src_a = jnp.pad(a2s_edge_index[0], pad_e)
    dst_a = jnp.pad(a2s_edge_index[1], pad_e)
    src_s = jnp.pad(s2s_edge_index[0], pad_e)
    dst_s = jnp.pad(s2s_edge_index[1], pad_e)
    disa = jnp.pad(a2s_dis, (pad_e, (0, 0)))
    diss = jnp.pad(s2s_dis, (pad_e, (0, 0)))
    tabs = [jnp.pad(t, ((0, _NP - _N), (0, 0))) for t in (tsa, tda, tss, tds)]

    w2bd = jnp.zeros((2 * _MLP, 2 * _MLP), f32)
    w2bd = w2bd.at[:_MLP, :_MLP].set(p["u2h_dis_W2"])
    w2bd = w2bd.at[_MLP:, _MLP:].set(p["h2h_dis_W2"])
    b2c = jnp.concatenate([p["u2h_dis_b2"], p["h2h_dis_b2"]]).reshape(1, -1)
    w3bd = jnp.zeros((2 * _MLP, 2 * _HID), f32)
    w3bd = w3bd.at[:_MLP, :_HID].set(p["u2h_dis_W3"])
    w3bd = w3bd.at[_MLP:, _HID:].set(p["h2h_dis_W3"])
    b3c = jnp.concatenate([p["u2h_dis_b3"], p["h2h_dis_b3"]]).reshape(1, -1)
    wvals = [p["u2h_dis_W1"][4:5], p["h2h_dis_W1"][4:5],
             w2bd, b2c, w3bd, b3c]
    wspecs = [_full((1, _MLP)), _full((1, _MLP)),
              _full((2 * _MLP, 2 * _MLP)), _full((1, 2 * _MLP)),
              _full((2 * _MLP, 2 * _HID)), _full((1, 2 * _HID))]

    edge_spec = [
        pl.BlockSpec((_BE, 128), lambda i: (i, 0)),
        pl.BlockSpec((_BE, 128), lambda i: (i, 0)),
        pl.BlockSpec((_BE, 1), lambda i: (i, 0)),
    ]

    parts = []
    for h0 in (0, _EPH):
        sl = slice(h0, h0 + _EPH)
        idxs = [a[sl].reshape(_EPH // _GSB, _GSB)
                for a in (src_a, dst_a, src_s, dst_s)]
        xsa, xda, xss, xds = _sc_gather4(tabs, idxs)
        msga, ems = pl.pallas_call(
            _make_edge_body(h0),
            grid=(_EPH // _BE,),
            in_specs=edge_spec + edge_spec + wspecs,
            out_specs=[
                pl.BlockSpec((_BE, 128), lambda i: (i, 0)),
                pl.BlockSpec((_BE, 128), lambda i: (i, 0)),
            ],
            out_shape=[
                jax.ShapeDtypeStruct((_EPH, 128), f32),
                jax.ShapeDtypeStruct((_EPH, 128), f32),
            ],
        )(xsa, xda, disa[sl], xss, xds, diss[sl], *wvals)
        parts.append(_sc_scatter(
            dst_a[sl].reshape(_EPH // _SB, _SB), msga,
            dst_s[sl].reshape(_EPH // _SB, _SB), ems))
    part1, part2 = parts

    cw3 = p["coef_W3"].reshape(_MLP, _HID, 3 * _HID)
    cw3 = jnp.pad(cw3, ((0, 0), (0, 0), (0, 128 - 3 * _HID)))
    cw3 = cw3.transpose(2, 0, 1).reshape(128, _MLP * _HID)
    cb3 = p["coef_b3"].reshape(_HID, 3 * _HID)
    cb3 = jnp.pad(cb3, ((0, 0), (0, 128 - 3 * _HID))).T
    sel = jnp.arange(_MLP * _HID)
    rsel = jnp.zeros((_MLP, _MLP * _HID), f32).at[sel // _HID, sel].set(1.0)
    fsel = jnp.zeros((128, _HID), f32).at[
        jnp.arange(128), jnp.arange(128) % _HID].set(1.0)

    out = pl.pallas_call(
        _final_body,
        grid=(_N // _BN,),
        in_specs=[
            pl.BlockSpec((_BN, _HID), lambda i: (i, 0)),
            pl.BlockSpec((2, _BN, 128), lambda i: (0, i, 0)),
            pl.BlockSpec((2, _BN, 128), lambda i: (0, i, 0)),
            pl.BlockSpec((_BN, 2), lambda i: (i, 0)),
            _full((2, _MLP)), _full((1, _MLP)),
            _full((_MLP, _MLP)), _full((1, _MLP)),
            _full((128, _MLP * _HID)), _full((128, _HID)),
            _full((_MLP, _MLP * _HID)), _full((128, _HID)),
            _full((2, _MLP)), _full((1, _MLP)),
            _full((_MLP, _MLP)), _full((1, _MLP)),
            _full((_MLP, _HID)), _full((1, _HID)),
        ],
        out_specs=[pl.BlockSpec((_BN, _HID), lambda i: (i, 0))],
        out_shape=[jax.ShapeDtypeStruct((_N, _HID), f32)],
    )(h, part1, part2, pos_state,
      p["coef_W1"], row(p["coef_b1"]), p["coef_W2"], row(p["coef_b2"]),
      cw3, cb3, rsel, fsel,
      p["const_W1"], row(p["const_b1"]), p["const_W2"], row(p["const_b2"]),
      p["const_W3"], row(p["const_b3"]))[0]
    return out
